# baseline (speedup 1.0000x reference)
import functools

import jax
import jax.numpy as jnp
from jax import lax
from jax.experimental import pallas as pl
from jax.experimental.pallas import tpu as pltpu
from jax.experimental.pallas import tpu_sc as plsc

_INFO = plsc.get_sparse_core_info()
_NC, _NS, _L = 1, 8, _INFO.num_lanes
_NW = _NC * _NS

_BATCH = 16384
_B_PER_W = _BATCH // _NW
_HALF = _B_PER_W // 2


def _sc_gather_sigmoid(items, table_1d):
    mesh = plsc.VectorSubcoreMesh(core_axis_name="c", subcore_axis_name="s", num_cores=1, num_subcores=8)

    @functools.partial(
        pl.kernel,
        mesh=mesh,
        out_type=jax.ShapeDtypeStruct((_BATCH,), jnp.float32),
        scratch_types=[
            pltpu.VMEM((_HALF,), jnp.int32),
            pltpu.VMEM((_HALF,), jnp.int32),
            pltpu.VMEM((_B_PER_W,), jnp.float32),
            pltpu.SemaphoreType.DMA,
            pltpu.SemaphoreType.DMA,
            pltpu.SemaphoreType.DMA,
            pltpu.SemaphoreType.DMA,
        ],
    )
    def k(items_hbm, table_hbm, out_hbm, idx1, idx2, vals_v, i1, i2, g1, g2):
        wid = lax.axis_index("s") * _NC + lax.axis_index("c")
        base = wid * _B_PER_W
        c1 = pltpu.async_copy(items_hbm.at[pl.ds(base, _HALF)], idx1, i1)
        c2 = pltpu.async_copy(items_hbm.at[pl.ds(base + _HALF, _HALF)], idx2, i2)
        c1.wait()
        d1 = pltpu.async_copy(table_hbm.at[idx1], vals_v.at[pl.ds(0, _HALF)], g1)
        c2.wait()
        d2 = pltpu.async_copy(
            table_hbm.at[idx2], vals_v.at[pl.ds(_HALF, _HALF)], g2
        )

        def sigmoid_chunk(i, carry):
            x = vals_v[pl.ds(i * _L, _L)]
            vals_v[pl.ds(i * _L, _L)] = 1.0 / (1.0 + jnp.exp(-x))
            return carry

        d1.wait()
        lax.fori_loop(0, _HALF // _L, sigmoid_chunk, 0, unroll=4)
        d2.wait()
        lax.fori_loop(_HALF // _L, _B_PER_W // _L, sigmoid_chunk, 0, unroll=4)
        pltpu.sync_copy(vals_v, out_hbm.at[pl.ds(base, _B_PER_W)])

    return k(items, table_1d)


def kernel(items, data_bias_weight):
    return _sc_gather_sigmoid(items, data_bias_weight.reshape(-1))


# speedup vs baseline: 1.0228x; 1.0228x over previous
import functools

import jax
import jax.numpy as jnp
from jax import lax
from jax.experimental import pallas as pl
from jax.experimental.pallas import tpu as pltpu
from jax.experimental.pallas import tpu_sc as plsc

_INFO = plsc.get_sparse_core_info()
_L = _INFO.num_lanes
_NW = 16

_BATCH = 16384
_B_PER_W = _BATCH // _NW
_HALF = _B_PER_W // 2


def _sc_gather_sigmoid(items, table_1d):
    mesh = plsc.VectorSubcoreMesh(
        core_axis_name="c", subcore_axis_name="s", num_cores=1
    )

    @functools.partial(
        pl.kernel,
        mesh=mesh,
        out_type=jax.ShapeDtypeStruct((_BATCH,), jnp.float32),
        scratch_types=[
            pltpu.VMEM((_HALF,), jnp.int32),
            pltpu.VMEM((_HALF,), jnp.int32),
            pltpu.VMEM((_B_PER_W,), jnp.float32),
            pltpu.SemaphoreType.DMA,
            pltpu.SemaphoreType.DMA,
            pltpu.SemaphoreType.DMA,
            pltpu.SemaphoreType.DMA,
            pltpu.SemaphoreType.DMA,
        ],
    )
    def k(items_hbm, table_hbm, out_hbm, idx1, idx2, vals_v, i1, i2, g1, g2, osem):
        wid = lax.axis_index("s")
        base = wid * _B_PER_W
        c1 = pltpu.async_copy(items_hbm.at[pl.ds(base, _HALF)], idx1, i1)
        c2 = pltpu.async_copy(items_hbm.at[pl.ds(base + _HALF, _HALF)], idx2, i2)
        c1.wait()
        d1 = pltpu.async_copy(table_hbm.at[idx1], vals_v.at[pl.ds(0, _HALF)], g1)
        c2.wait()
        d2 = pltpu.async_copy(
            table_hbm.at[idx2], vals_v.at[pl.ds(_HALF, _HALF)], g2
        )

        def sigmoid_chunk(i, carry):
            x = vals_v[pl.ds(i * _L, _L)]
            vals_v[pl.ds(i * _L, _L)] = 1.0 / (1.0 + jnp.exp(-x))
            return carry

        d1.wait()
        lax.fori_loop(0, _HALF // _L, sigmoid_chunk, 0, unroll=8)
        o1 = pltpu.async_copy(
            vals_v.at[pl.ds(0, _HALF)], out_hbm.at[pl.ds(base, _HALF)], osem
        )
        d2.wait()
        lax.fori_loop(_HALF // _L, _B_PER_W // _L, sigmoid_chunk, 0, unroll=8)
        o2 = pltpu.async_copy(
            vals_v.at[pl.ds(_HALF, _HALF)],
            out_hbm.at[pl.ds(base + _HALF, _HALF)],
            osem,
        )
        o1.wait()
        o2.wait()

    return k(items, table_1d)


def kernel(items, data_bias_weight):
    return _sc_gather_sigmoid(items, data_bias_weight.reshape(-1))
